# baseline (speedup 1.0000x reference)
import functools

import jax
import jax.numpy as jnp
from jax import lax
from jax.experimental import pallas as pl
from jax.experimental.pallas import tpu as pltpu
from jax.experimental.pallas import tpu_sc as plsc

N_CLUSTERS = 512
D = 32
B = 16
S = 4096
N_TOKENS = B * S


def _argmin_body(xt_ref, ct_ref, idx_ref):
    xb = xt_ref[0]
    ct = ct_ref[...]
    xc_t = lax.dot_general(
        ct, xb, (((0,), (0,)), ((), ())),
        preferred_element_type=jnp.float32,
        precision=lax.Precision.DEFAULT)
    hc2 = 0.5 * jnp.sum(ct * ct, axis=0)
    t = hc2[:, None] - xc_t
    idx_ref[...] = jnp.argmin(t, axis=0).astype(jnp.int32)


def _nearest_idx(xt, ct):
    return pl.pallas_call(
        _argmin_body,
        grid=(B,),
        in_specs=[
            pl.BlockSpec((1, D, S), lambda i: (i, 0, 0)),
            pl.BlockSpec((D, N_CLUSTERS), lambda i: (0, 0)),
        ],
        out_specs=pl.BlockSpec((S,), lambda i: (i,)),
        out_shape=jax.ShapeDtypeStruct((N_TOKENS,), jnp.int32),
    )(xt, ct)


_NC, _NS, _L = 2, 16, 16
_NW = _NC * _NS
_BPW = N_TOKENS // _NW
_NG = _BPW // _L


@functools.lru_cache(maxsize=None)
def _gather_fn():
    mesh = plsc.VectorSubcoreMesh(
        core_axis_name="c", subcore_axis_name="s",
        num_cores=_NC, num_subcores=_NS)

    @functools.partial(
        pl.kernel,
        mesh=mesh,
        out_type=jax.ShapeDtypeStruct((B, D, S), jnp.float32),
        scratch_types=[
            pltpu.VMEM((N_CLUSTERS * D,), jnp.float32),
            pltpu.VMEM((_BPW,), jnp.int32),
            pltpu.VMEM((D, _BPW), jnp.float32),
            pltpu.SemaphoreType.DMA,
        ],
        compiler_params=pltpu.CompilerParams(
            use_tc_tiling_on_sc=True, needs_layout_passes=False),
    )
    def _gather_rows(tablet_hbm, idx_hbm, out_hbm, tablet_v, idx_v, rows_v,
                     sem):
        wid = lax.axis_index("s") * _NC + lax.axis_index("c")
        pltpu.sync_copy(tablet_hbm, tablet_v)
        pltpu.sync_copy(idx_hbm.at[pl.ds(wid * _BPW, _BPW)], idx_v)

        b = wid // 2
        s0 = (wid % 2) * _BPW
        nch = 4
        chw = _BPW // nch

        copies = []
        for h in range(nch):
            @plsc.parallel_loop(h * _NG // nch, (h + 1) * _NG // nch,
                                unroll=2)
            def body(g):
                idx16 = idx_v[pl.ds(g * _L, _L)]
                for d in range(D):
                    vals = plsc.load_gather(
                        tablet_v, [idx16 + (d * N_CLUSTERS)])
                    rows_v[d, pl.ds(g * _L, _L)] = vals

            copies.append(pltpu.async_copy(
                rows_v.at[pl.ds(0, D), pl.ds(h * chw, chw)],
                out_hbm.at[b, pl.ds(0, D), pl.ds(s0 + h * chw, chw)],
                sem))
        for cp in copies:
            cp.wait()

    return _gather_rows


def kernel(x, cluster_centers):
    xt = jnp.swapaxes(x, 1, 2)
    ct = jnp.swapaxes(cluster_centers, 0, 1)
    idx = _nearest_idx(xt, ct)
    outt = _gather_fn()(ct.reshape(N_CLUSTERS * D), idx)
    return jnp.swapaxes(outt, 1, 2)


# speedup vs baseline: 1.0422x; 1.0422x over previous
import functools

import jax
import jax.numpy as jnp
from jax import lax
from jax.experimental import pallas as pl
from jax.experimental.pallas import tpu as pltpu
from jax.experimental.pallas import tpu_sc as plsc

N_CLUSTERS = 512
D = 32
B = 16
S = 4096
N_TOKENS = B * S


_BB = 2


def _argmin_body(xt_ref, ct_ref, idx_ref):
    ct = ct_ref[...]
    hc2 = 0.5 * jnp.sum(ct * ct, axis=0)
    for j in range(_BB):
        xb = xt_ref[j]
        xc_t = lax.dot_general(
            ct, xb, (((0,), (0,)), ((), ())),
            preferred_element_type=jnp.float32,
            precision=lax.Precision.DEFAULT)
        t = hc2[:, None] - xc_t
        idx_ref[pl.ds(j * S, S)] = jnp.argmin(t, axis=0).astype(jnp.int32)


def _nearest_idx(xt, ct):
    return pl.pallas_call(
        _argmin_body,
        grid=(B // _BB,),
        in_specs=[
            pl.BlockSpec((_BB, D, S), lambda i: (i, 0, 0)),
            pl.BlockSpec((D, N_CLUSTERS), lambda i: (0, 0)),
        ],
        out_specs=pl.BlockSpec((_BB * S,), lambda i: (i,)),
        out_shape=jax.ShapeDtypeStruct((N_TOKENS,), jnp.int32),
    )(xt, ct)


_NC, _NS, _L = 2, 16, 16
_NW = _NC * _NS
_BPW = N_TOKENS // _NW
_NG = _BPW // _L


@functools.lru_cache(maxsize=None)
def _gather_fn():
    mesh = plsc.VectorSubcoreMesh(
        core_axis_name="c", subcore_axis_name="s",
        num_cores=_NC, num_subcores=_NS)

    @functools.partial(
        pl.kernel,
        mesh=mesh,
        out_type=jax.ShapeDtypeStruct((B, D, S), jnp.float32),
        scratch_types=[
            pltpu.VMEM((N_CLUSTERS * D,), jnp.float32),
            pltpu.VMEM((_BPW,), jnp.int32),
            pltpu.VMEM((D, _BPW), jnp.float32),
            pltpu.SemaphoreType.DMA,
        ],
        compiler_params=pltpu.CompilerParams(
            use_tc_tiling_on_sc=True, needs_layout_passes=False),
    )
    def _gather_rows(tablet_hbm, idx_hbm, out_hbm, tablet_v, idx_v, rows_v,
                     sem):
        wid = lax.axis_index("s") * _NC + lax.axis_index("c")
        pltpu.sync_copy(tablet_hbm, tablet_v)
        pltpu.sync_copy(idx_hbm.at[pl.ds(wid * _BPW, _BPW)], idx_v)

        b = wid // 2
        s0 = (wid % 2) * _BPW
        nch = 2
        chw = _BPW // nch

        copies = []
        for h in range(nch):
            @plsc.parallel_loop(h * _NG // nch, (h + 1) * _NG // nch,
                                unroll=2)
            def body(g):
                idx16 = idx_v[pl.ds(g * _L, _L)]
                for d in range(D):
                    vals = plsc.load_gather(
                        tablet_v, [idx16 + (d * N_CLUSTERS)])
                    rows_v[d, pl.ds(g * _L, _L)] = vals

            copies.append(pltpu.async_copy(
                rows_v.at[pl.ds(0, D), pl.ds(h * chw, chw)],
                out_hbm.at[b, pl.ds(0, D), pl.ds(s0 + h * chw, chw)],
                sem))
        for cp in copies:
            cp.wait()

    return _gather_rows


def kernel(x, cluster_centers):
    xt = jnp.swapaxes(x, 1, 2)
    ct = jnp.swapaxes(cluster_centers, 0, 1)
    idx = _nearest_idx(xt, ct)
    outt = _gather_fn()(ct.reshape(N_CLUSTERS * D), idx)
    return jnp.swapaxes(outt, 1, 2)


# speedup vs baseline: 1.0513x; 1.0088x over previous
import functools

import jax
import jax.numpy as jnp
from jax import lax
from jax.experimental import pallas as pl
from jax.experimental.pallas import tpu as pltpu
from jax.experimental.pallas import tpu_sc as plsc

N_CLUSTERS = 512
D = 32
B = 16
S = 4096
N_TOKENS = B * S


_BB = 4


def _argmin_body(xt_ref, ct_ref, idx_ref):
    ct = ct_ref[...]
    hc2 = 0.5 * jnp.sum(ct * ct, axis=0)
    for j in range(_BB):
        xb = xt_ref[j]
        xc_t = lax.dot_general(
            ct, xb, (((0,), (0,)), ((), ())),
            preferred_element_type=jnp.float32,
            precision=lax.Precision.DEFAULT)
        t = hc2[:, None] - xc_t
        idx_ref[pl.ds(j * S, S)] = jnp.argmin(t, axis=0).astype(jnp.int32)


def _nearest_idx(xt, ct):
    return pl.pallas_call(
        _argmin_body,
        grid=(B // _BB,),
        in_specs=[
            pl.BlockSpec((_BB, D, S), lambda i: (i, 0, 0)),
            pl.BlockSpec((D, N_CLUSTERS), lambda i: (0, 0)),
        ],
        out_specs=pl.BlockSpec((_BB * S,), lambda i: (i,)),
        out_shape=jax.ShapeDtypeStruct((N_TOKENS,), jnp.int32),
    )(xt, ct)


_NC, _NS, _L = 2, 16, 16
_NW = _NC * _NS
_BPW = N_TOKENS // _NW
_NG = _BPW // _L


@functools.lru_cache(maxsize=None)
def _gather_fn():
    mesh = plsc.VectorSubcoreMesh(
        core_axis_name="c", subcore_axis_name="s",
        num_cores=_NC, num_subcores=_NS)

    @functools.partial(
        pl.kernel,
        mesh=mesh,
        out_type=jax.ShapeDtypeStruct((B, D, S), jnp.float32),
        scratch_types=[
            pltpu.VMEM((N_CLUSTERS * D,), jnp.float32),
            pltpu.VMEM((_BPW,), jnp.int32),
            pltpu.VMEM((D, _BPW), jnp.float32),
            pltpu.SemaphoreType.DMA,
        ],
        compiler_params=pltpu.CompilerParams(
            use_tc_tiling_on_sc=True, needs_layout_passes=False),
    )
    def _gather_rows(tablet_hbm, idx_hbm, out_hbm, tablet_v, idx_v, rows_v,
                     sem):
        wid = lax.axis_index("s") * _NC + lax.axis_index("c")
        pltpu.sync_copy(tablet_hbm, tablet_v)
        pltpu.sync_copy(idx_hbm.at[pl.ds(wid * _BPW, _BPW)], idx_v)

        b = wid // 2
        s0 = (wid % 2) * _BPW
        nch = 2
        chw = _BPW // nch

        copies = []
        for h in range(nch):
            @plsc.parallel_loop(h * _NG // nch, (h + 1) * _NG // nch,
                                unroll=2)
            def body(g):
                idx16 = idx_v[pl.ds(g * _L, _L)]
                for d in range(D):
                    vals = plsc.load_gather(
                        tablet_v, [idx16 + (d * N_CLUSTERS)])
                    rows_v[d, pl.ds(g * _L, _L)] = vals

            copies.append(pltpu.async_copy(
                rows_v.at[pl.ds(0, D), pl.ds(h * chw, chw)],
                out_hbm.at[b, pl.ds(0, D), pl.ds(s0 + h * chw, chw)],
                sem))
        for cp in copies:
            cp.wait()

    return _gather_rows


def kernel(x, cluster_centers):
    xt = jnp.swapaxes(x, 1, 2)
    ct = jnp.swapaxes(cluster_centers, 0, 1)
    idx = _nearest_idx(xt, ct)
    outt = _gather_fn()(ct.reshape(N_CLUSTERS * D), idx)
    return jnp.swapaxes(outt, 1, 2)


# speedup vs baseline: 1.0528x; 1.0014x over previous
import functools

import jax
import jax.numpy as jnp
from jax import lax
from jax.experimental import pallas as pl
from jax.experimental.pallas import tpu as pltpu
from jax.experimental.pallas import tpu_sc as plsc

N_CLUSTERS = 512
D = 32
B = 16
S = 4096
N_TOKENS = B * S


_BB = 8


def _argmin_body(xt_ref, ct_ref, idx_ref):
    ct = ct_ref[...]
    hc2 = 0.5 * jnp.sum(ct * ct, axis=0)
    for j in range(_BB):
        xb = xt_ref[j]
        xc_t = lax.dot_general(
            ct, xb, (((0,), (0,)), ((), ())),
            preferred_element_type=jnp.float32,
            precision=lax.Precision.DEFAULT)
        t = hc2[:, None] - xc_t
        idx_ref[pl.ds(j * S, S)] = jnp.argmin(t, axis=0).astype(jnp.int32)


def _nearest_idx(xt, ct):
    return pl.pallas_call(
        _argmin_body,
        grid=(B // _BB,),
        in_specs=[
            pl.BlockSpec((_BB, D, S), lambda i: (i, 0, 0)),
            pl.BlockSpec((D, N_CLUSTERS), lambda i: (0, 0)),
        ],
        out_specs=pl.BlockSpec((_BB * S,), lambda i: (i,)),
        out_shape=jax.ShapeDtypeStruct((N_TOKENS,), jnp.int32),
    )(xt, ct)


_NC, _NS, _L = 2, 16, 16
_NW = _NC * _NS
_BPW = N_TOKENS // _NW
_NG = _BPW // _L


@functools.lru_cache(maxsize=None)
def _gather_fn():
    mesh = plsc.VectorSubcoreMesh(
        core_axis_name="c", subcore_axis_name="s",
        num_cores=_NC, num_subcores=_NS)

    @functools.partial(
        pl.kernel,
        mesh=mesh,
        out_type=jax.ShapeDtypeStruct((B, D, S), jnp.float32),
        scratch_types=[
            pltpu.VMEM((N_CLUSTERS * D,), jnp.float32),
            pltpu.VMEM((_BPW,), jnp.int32),
            pltpu.VMEM((D, _BPW), jnp.float32),
            pltpu.SemaphoreType.DMA,
        ],
        compiler_params=pltpu.CompilerParams(
            use_tc_tiling_on_sc=True, needs_layout_passes=False),
    )
    def _gather_rows(tablet_hbm, idx_hbm, out_hbm, tablet_v, idx_v, rows_v,
                     sem):
        wid = lax.axis_index("s") * _NC + lax.axis_index("c")
        pltpu.sync_copy(tablet_hbm, tablet_v)
        pltpu.sync_copy(idx_hbm.at[pl.ds(wid * _BPW, _BPW)], idx_v)

        b = wid // 2
        s0 = (wid % 2) * _BPW
        nch = 2
        chw = _BPW // nch

        copies = []
        for h in range(nch):
            @plsc.parallel_loop(h * _NG // nch, (h + 1) * _NG // nch,
                                unroll=2)
            def body(g):
                idx16 = idx_v[pl.ds(g * _L, _L)]
                for d in range(D):
                    vals = plsc.load_gather(
                        tablet_v, [idx16 + (d * N_CLUSTERS)])
                    rows_v[d, pl.ds(g * _L, _L)] = vals

            copies.append(pltpu.async_copy(
                rows_v.at[pl.ds(0, D), pl.ds(h * chw, chw)],
                out_hbm.at[b, pl.ds(0, D), pl.ds(s0 + h * chw, chw)],
                sem))
        for cp in copies:
            cp.wait()

    return _gather_rows


def kernel(x, cluster_centers):
    xt = jnp.swapaxes(x, 1, 2)
    ct = jnp.swapaxes(cluster_centers, 0, 1)
    idx = _nearest_idx(xt, ct)
    outt = _gather_fn()(ct.reshape(N_CLUSTERS * D), idx)
    return jnp.swapaxes(outt, 1, 2)


# speedup vs baseline: 1.0533x; 1.0004x over previous
import functools

import jax
import jax.numpy as jnp
from jax import lax
from jax.experimental import pallas as pl
from jax.experimental.pallas import tpu as pltpu
from jax.experimental.pallas import tpu_sc as plsc

N_CLUSTERS = 512
D = 32
B = 16
S = 4096
N_TOKENS = B * S


_BB = 4


def _argmin_body(xt_ref, ct_ref, idx_ref):
    ct = ct_ref[...]
    hc2 = 0.5 * jnp.sum(ct * ct, axis=0)
    for j in range(_BB):
        xb = xt_ref[j]
        xc_t = lax.dot_general(
            ct, xb, (((0,), (0,)), ((), ())),
            preferred_element_type=jnp.float32,
            precision=lax.Precision.DEFAULT)
        t = hc2[:, None] - xc_t
        idx_ref[pl.ds(j * S, S)] = jnp.argmin(t, axis=0).astype(jnp.int32)


def _nearest_idx(xt, ct):
    return pl.pallas_call(
        _argmin_body,
        grid=(B // _BB,),
        in_specs=[
            pl.BlockSpec((_BB, D, S), lambda i: (i, 0, 0)),
            pl.BlockSpec((D, N_CLUSTERS), lambda i: (0, 0)),
        ],
        out_specs=pl.BlockSpec((_BB * S,), lambda i: (i,)),
        out_shape=jax.ShapeDtypeStruct((N_TOKENS,), jnp.int32),
    )(xt, ct)


_NC, _NS, _L = 2, 16, 16
_NW = _NC * _NS
_BPW = N_TOKENS // _NW
_NG = _BPW // _L


@functools.lru_cache(maxsize=None)
def _gather_fn():
    mesh = plsc.VectorSubcoreMesh(
        core_axis_name="c", subcore_axis_name="s",
        num_cores=_NC, num_subcores=_NS)

    @functools.partial(
        pl.kernel,
        mesh=mesh,
        out_type=jax.ShapeDtypeStruct((B, D, S), jnp.float32),
        scratch_types=[
            pltpu.VMEM((N_CLUSTERS * D,), jnp.float32),
            pltpu.VMEM((_BPW,), jnp.int32),
            pltpu.VMEM((D, _BPW), jnp.float32),
            pltpu.SemaphoreType.DMA,
        ],
        compiler_params=pltpu.CompilerParams(
            use_tc_tiling_on_sc=True, needs_layout_passes=False),
    )
    def _gather_rows(tablet_hbm, idx_hbm, out_hbm, tablet_v, idx_v, rows_v,
                     sem):
        wid = lax.axis_index("s") * _NC + lax.axis_index("c")
        pltpu.sync_copy(tablet_hbm, tablet_v)
        pltpu.sync_copy(idx_hbm.at[pl.ds(wid * _BPW, _BPW)], idx_v)

        b = wid // 2
        s0 = (wid % 2) * _BPW
        nch = 2
        chw = _BPW // nch

        copies = []
        for h in range(nch):
            @plsc.parallel_loop(h * _NG // nch, (h + 1) * _NG // nch,
                                unroll=2)
            def body(g):
                idx16 = idx_v[pl.ds(g * _L, _L)]
                for d in range(D):
                    vals = plsc.load_gather(
                        tablet_v, [idx16 + (d * N_CLUSTERS)])
                    rows_v[d, pl.ds(g * _L, _L)] = vals

            copies.append(pltpu.async_copy(
                rows_v.at[pl.ds(0, D), pl.ds(h * chw, chw)],
                out_hbm.at[b, pl.ds(0, D), pl.ds(s0 + h * chw, chw)],
                sem))
        for cp in copies:
            cp.wait()

    return _gather_rows


def kernel(x, cluster_centers):
    xt = jnp.swapaxes(x, 1, 2)
    ct = jnp.swapaxes(cluster_centers, 0, 1)
    idx = _nearest_idx(xt, ct)
    outt = _gather_fn()(ct.reshape(N_CLUSTERS * D), idx)
    return jnp.swapaxes(outt, 1, 2)
